# baseline (speedup 1.0000x reference)
import functools

import jax
import jax.numpy as jnp
from jax import lax
from jax.experimental import pallas as pl
from jax.experimental.pallas import tpu as pltpu
from jax.experimental.pallas import tpu_sc as plsc

NC = 2
NS = 16
NW = NC * NS
CH = 80
U = 10
UA = 5

_f32 = jnp.float32


def _round_up(v, m):
    return (v + m - 1) // m * m


def _make_degree_kernel(n_pad, e_pad):
    per_w = e_pad // NW
    n_ch = per_w // CH
    rows_pt = n_pad // NS
    mesh = plsc.VectorSubcoreMesh(core_axis_name="c", subcore_axis_name="s")
    n_wave = n_ch // U
    tail = n_ch - n_wave * U

    @functools.partial(
        pl.kernel,
        out_type=jax.ShapeDtypeStruct((NC, 1, n_pad), _f32),
        mesh=mesh,
        compiler_params=pltpu.CompilerParams(use_tc_tiling_on_sc=False),
        scratch_types=[
            pltpu.VMEM((n_ch, CH), jnp.int32),
            pltpu.VMEM((CH,), _f32),
            pltpu.VMEM((rows_pt,), _f32),
            pltpu.VMEM_SHARED((n_pad,), _f32),
            pltpu.SemaphoreType.DMA,
        ],
    )
    def deg_kernel(ei_hbm, out_hbm, didx2, ones_v, stage, acc, ssem):
        cid = lax.axis_index("c")
        sid = lax.axis_index("s")
        wid = sid * NC + cid

        pltpu.sync_copy(ei_hbm.at[1, pl.ds(wid * n_ch, n_ch)], didx2)

        def _fill(i, _):
            ones_v[pl.ds(i * 16, 16)] = jnp.ones((16,), _f32)
            return 0
        lax.fori_loop(0, CH // 16, _fill, 0)

        def _zero(i, _):
            stage[pl.ds(i * 16, 16)] = jnp.zeros((16,), _f32)
            return 0
        lax.fori_loop(0, rows_pt // 16, _zero, 0)
        pltpu.sync_copy(stage, acc.at[pl.ds(sid * rows_pt, rows_pt)])
        plsc.subcore_barrier()

        def _wave(wv, _):
            base = wv * U
            descs = [pltpu.async_copy(ones_v, acc.at[didx2.at[base + b]],
                                      ssem, add=True)
                     for b in range(U)]
            for d in descs:
                d.wait()
            return 0
        lax.fori_loop(0, n_wave, _wave, 0)
        if tail:
            descs = [pltpu.async_copy(ones_v,
                                      acc.at[didx2.at[n_wave * U + b]],
                                      ssem, add=True)
                     for b in range(tail)]
            for d in descs:
                d.wait()
        plsc.subcore_barrier()

        pltpu.sync_copy(acc.at[pl.ds(sid * rows_pt, rows_pt)], stage)
        pltpu.sync_copy(stage,
                        out_hbm.at[cid, 0, pl.ds(sid * rows_pt, rows_pt)])

    return deg_kernel


def _make_agg_kernel(n_pad, e_pad, width):
    per_w = e_pad // NW
    n_ch = per_w // CH
    rows_pt = n_pad // NS
    n_wave = n_ch // UA
    tail = n_ch - n_wave * UA
    n_pair = (n_wave - 1) // 2
    rem = n_wave - 2 * n_pair
    mesh = plsc.VectorSubcoreMesh(core_axis_name="c", subcore_axis_name="s")

    @functools.partial(
        pl.kernel,
        out_type=jax.ShapeDtypeStruct((NC, n_pad, width), _f32),
        mesh=mesh,
        compiler_params=pltpu.CompilerParams(use_tc_tiling_on_sc=False),
        scratch_types=[
            pltpu.VMEM((n_ch, CH), jnp.int32),
            pltpu.VMEM((n_ch, CH), jnp.int32),
            pltpu.VMEM((2, UA, CH, width), _f32),
            pltpu.VMEM((rows_pt, width), _f32),
            pltpu.VMEM_SHARED((n_pad, width), _f32),
            pltpu.SemaphoreType.DMA,
            pltpu.SemaphoreType.DMA,
            pltpu.SemaphoreType.DMA,
        ],
    )
    def agg_kernel(g_hbm, ei_hbm, out_hbm,
                   sidx2, didx2, rows, stage, acc, gsem0, gsem1, ssem):
        cid = lax.axis_index("c")
        sid = lax.axis_index("s")
        wid = sid * NC + cid
        gsems = (gsem0, gsem1)

        pltpu.sync_copy(ei_hbm.at[0, pl.ds(wid * n_ch, n_ch)], sidx2)
        pltpu.sync_copy(ei_hbm.at[1, pl.ds(wid * n_ch, n_ch)], didx2)

        def _zero(i, _):
            def _zcol(j, _):
                stage[i, pl.ds(j * 16, 16)] = jnp.zeros((16,), _f32)
                return 0
            lax.fori_loop(0, width // 16, _zcol, 0)
            return 0
        lax.fori_loop(0, rows_pt, _zero, 0)
        pltpu.sync_copy(stage, acc.at[pl.ds(sid * rows_pt, rows_pt)])
        plsc.subcore_barrier()

        def _fire_g(wv, bank, count=UA):
            return [pltpu.async_copy(g_hbm.at[sidx2.at[wv * UA + b]],
                                     rows.at[bank, b], gsems[bank])
                    for b in range(count)]

        def _drain_consume(wv, bank, count=UA):
            for b in range(count):
                pltpu.make_async_copy(g_hbm.at[sidx2.at[wv * UA + b]],
                                      rows.at[bank, b], gsems[bank]).wait()
            sds = [pltpu.async_copy(rows.at[bank, b],
                                    acc.at[didx2.at[wv * UA + b]],
                                    ssem, add=True)
                   for b in range(count)]
            for d in sds:
                d.wait()

        _fire_g(0, 0)

        def _pair(i, _):
            wa = 2 * i
            _fire_g(wa + 1, 1)
            _drain_consume(wa, 0)
            _fire_g(wa + 2, 0)
            _drain_consume(wa + 1, 1)
            return 0
        lax.fori_loop(0, n_pair, _pair, 0)

        if rem == 2:
            wa = 2 * n_pair
            _fire_g(wa + 1, 1)
            _drain_consume(wa, 0)
            _drain_consume(wa + 1, 1)
        else:
            _drain_consume(2 * n_pair, 0)
        if tail:
            base = n_wave * UA
            gds = [pltpu.async_copy(g_hbm.at[sidx2.at[base + b]],
                                    rows.at[0, b], gsem0)
                   for b in range(tail)]
            for d in gds:
                d.wait()
            sds = [pltpu.async_copy(rows.at[0, b],
                                    acc.at[didx2.at[base + b]],
                                    ssem, add=True)
                   for b in range(tail)]
            for d in sds:
                d.wait()
        plsc.subcore_barrier()

        pltpu.sync_copy(acc.at[pl.ds(sid * rows_pt, rows_pt)], stage)
        pltpu.sync_copy(stage, out_hbm.at[cid, pl.ds(sid * rows_pt, rows_pt)])

    return agg_kernel


def _mm(x, w1):
    n = x.shape[0]
    hid = w1.shape[1]

    def body(x_ref, w_ref, p_ref):
        p_ref[...] = jnp.dot(x_ref[...], w_ref[...],
                             preferred_element_type=_f32)

    return pl.pallas_call(
        body,
        out_shape=jax.ShapeDtypeStruct((n, hid), _f32),
    )(x, w1)


def _prep(p1, ccol):
    n, hid = p1.shape

    def body(p_ref, c_ref, g_ref, sp_ref):
        deg = c_ref[...]
        dis = lax.rsqrt(deg)
        inv = 1.0 / deg
        p = p_ref[...]
        g_ref[...] = p * dis
        sp_ref[...] = p * inv

    return pl.pallas_call(
        body,
        out_shape=(
            jax.ShapeDtypeStruct((n, hid), _f32),
            jax.ShapeDtypeStruct((n, hid), _f32),
        ),
    )(p1, ccol)


def _mid(s1, sp1, b1, ccol):
    n, hid = sp1.shape

    def body(s_ref, sp_ref, bias_ref, c_ref, g2_ref, sh_ref):
        deg = c_ref[...]
        dis = lax.rsqrt(deg)
        inv = 1.0 / deg
        s = s_ref[0, :n, :] + s_ref[1, :n, :]
        h = s * dis + sp_ref[...] + bias_ref[...]
        h = jnp.maximum(h, 0.0)
        g2_ref[...] = h * dis
        sh_ref[...] = h * inv

    return pl.pallas_call(
        body,
        out_shape=(
            jax.ShapeDtypeStruct((n, hid), _f32),
            jax.ShapeDtypeStruct((n, hid), _f32),
        ),
    )(s1, sp1, b1, ccol)


def _fin(s2, sh, ccol, wmu, bmu, wls, bls):
    n = sh.shape[0]
    out_d = wmu.shape[1]

    def body(s_ref, sh_ref, c_ref, wmu_ref, bmu_ref,
             wls_ref, bls_ref, mu_ref, ls_ref):
        dis = lax.rsqrt(c_ref[...])
        s = s_ref[0, :n, :] + s_ref[1, :n, :]
        agg = s * dis + sh_ref[...]
        mu_ref[...] = jnp.dot(agg, wmu_ref[...],
                              preferred_element_type=_f32) + bmu_ref[...]
        ls_ref[...] = jnp.dot(agg, wls_ref[...],
                              preferred_element_type=_f32) + bls_ref[...]

    return pl.pallas_call(
        body,
        out_shape=(
            jax.ShapeDtypeStruct((n, out_d), _f32),
            jax.ShapeDtypeStruct((n, out_d), _f32),
        ),
    )(s2, sh, ccol, wmu, bmu, wls, bls)


def kernel(x, edge_index, W1, b1, W_mu, b_mu, W_ls, b_ls):
    n, _ = x.shape
    hid = W1.shape[1]
    out_d = W_mu.shape[1]
    e = edge_index.shape[1]

    n_pad = _round_up(n + 1, NS * 128)
    e_pad = _round_up(e, NW * CH)

    if e_pad != e:
        pad = e_pad - e
        filler = jnp.stack([jnp.zeros((pad,), jnp.int32),
                            jnp.full((pad,), n, jnp.int32)])
        ei = jnp.concatenate([edge_index, filler], axis=1)
    else:
        ei = edge_index
    ei = ei.reshape(2, e_pad // CH, CH)

    deg_k = _make_degree_kernel(n_pad, e_pad)
    agg_k = _make_agg_kernel(n_pad, e_pad, hid)

    p1 = _mm(x, W1)
    cnt = deg_k(ei)
    ccol = (cnt[0, 0, :n] + cnt[1, 0, :n] + 1.0).reshape(n, 1)

    g1, sp1 = _prep(p1, ccol)
    s1 = agg_k(g1, ei)
    g2, sh = _mid(s1, sp1, b1.reshape(1, hid), ccol)
    s2 = agg_k(g2, ei)
    mu, ls = _fin(s2, sh, ccol, W_mu, b_mu.reshape(1, out_d),
                  W_ls, b_ls.reshape(1, out_d))
    return (mu, ls)
